# Initial kernel scaffold; baseline (speedup 1.0000x reference)
import functools

import jax
import jax.numpy as jnp
from jax import lax
from jax.experimental import pallas as pl
from jax.experimental.pallas import tpu as pltpu
from jax.experimental.pallas import tpu_sc as plsc

N, E, NG = 10000, 320000, 128
H, HP = 76, 80
DOUT = 64
NC, NS, NW = 2, 16, 32
EPW = E // NW
K = 400
KQ = K // 80
NCHUNK = EPW // K
RPT = N // NS

_f32 = jnp.float32


def _dot(a, b):
    return jnp.dot(a, b, preferred_element_type=_f32)


def _node_prologue(x, Wex, bex, Wni_p, bni_p, WeA, WeB, beA):
    BN = 2000

    def body(x_r, wex_r, bex_r, wni_r, bni_r, wea_r, web_r, bea_r,
             h_o, ha_o, hb_o):
        hx = jnp.maximum(_dot(x_r[...], wex_r[...]) + bex_r[...], 0.0)
        h = jnp.maximum(_dot(hx, wni_r[...]) + bni_r[...], 0.0)
        h_o[...] = h
        ha_o[...] = _dot(h, wea_r[...]) + bea_r[...]
        hb_o[...] = _dot(h, web_r[...])

    full = lambda s: pl.BlockSpec(s, lambda i: (0,) * len(s))
    return pl.pallas_call(
        body,
        grid=(N // BN,),
        in_specs=[
            pl.BlockSpec((BN, 128), lambda i: (i, 0)),
            full((128, 128)), full((1, 128)),
            full((128, HP)), full((1, HP)),
            full((HP, HP)), full((HP, HP)), full((1, HP)),
        ],
        out_specs=[pl.BlockSpec((BN, HP), lambda i: (i, 0))] * 3,
        out_shape=[jax.ShapeDtypeStruct((N, HP), _f32)] * 3,
        compiler_params=pltpu.CompilerParams(dimension_semantics=("parallel",)),
    )(x, Wex, bex, Wni_p, bni_p, WeA, WeB, beA)


def _edge_prologue(edge_attr, Wee, bee, Wei_p, bei_p, WeC0):
    BE = 8000

    def body(ea_r, wee_r, bee_r, wei_r, bei_r, wc_r, p_o):
        he = jnp.maximum(_dot(ea_r[...], wee_r[...]) + bee_r[...], 0.0)
        e0 = jnp.maximum(_dot(he, wei_r[...]) + bei_r[...], 0.0)
        p_o[...] = _dot(e0, wc_r[...])

    full = lambda s: pl.BlockSpec(s, lambda i: (0,) * len(s))
    return pl.pallas_call(
        body,
        grid=(E // BE,),
        in_specs=[
            pl.BlockSpec((BE, 16), lambda i: (i, 0)),
            full((16, 128)), full((1, 128)),
            full((128, HP)), full((1, HP)), full((HP, HP)),
        ],
        out_specs=pl.BlockSpec((BE, HP), lambda i: (i, 0)),
        out_shape=jax.ShapeDtypeStruct((E, HP), _f32),
        compiler_params=pltpu.CompilerParams(dimension_semantics=("parallel",)),
    )(edge_attr, Wee, bee, Wei_p, bei_p, WeC0)


def _p_matmul(e, WeC):
    BE = 8000

    def body(e_r, w_r, p_o):
        p_o[...] = _dot(e_r[...], w_r[...])

    return pl.pallas_call(
        body,
        grid=(E // BE,),
        in_specs=[
            pl.BlockSpec((BE, HP), lambda i: (i, 0)),
            pl.BlockSpec((HP, HP), lambda i: (0, 0)),
        ],
        out_specs=pl.BlockSpec((BE, HP), lambda i: (i, 0)),
        out_shape=jax.ShapeDtypeStruct((E, HP), _f32),
        compiler_params=pltpu.CompilerParams(dimension_semantics=("parallel",)),
    )(e, WeC)


def _node_update(h, m, WnA, WnB, bn1, WeA, WeB, beA):
    BN = 2000

    def body(h_r, m_r, wna_r, wnb_r, bn_r, wea_r, web_r, bea_r,
             h_o, ha_o, hb_o):
        msum = m_r[0] + m_r[1]
        hn = jnp.maximum(
            _dot(h_r[...], wna_r[...]) + _dot(msum, wnb_r[...]) + bn_r[...],
            0.0)
        h_o[...] = hn
        ha_o[...] = _dot(hn, wea_r[...]) + bea_r[...]
        hb_o[...] = _dot(hn, web_r[...])

    full = lambda s: pl.BlockSpec(s, lambda i: (0,) * len(s))
    return pl.pallas_call(
        body,
        grid=(N // BN,),
        in_specs=[
            pl.BlockSpec((BN, HP), lambda i: (i, 0)),
            pl.BlockSpec((NC, BN, HP), lambda i: (0, i, 0)),
            full((HP, HP)), full((HP, HP)), full((1, HP)),
            full((HP, HP)), full((HP, HP)), full((1, HP)),
        ],
        out_specs=[pl.BlockSpec((BN, HP), lambda i: (i, 0))] * 3,
        out_shape=[jax.ShapeDtypeStruct((N, HP), _f32)] * 3,
        compiler_params=pltpu.CompilerParams(dimension_semantics=("parallel",)),
    )(h, m, WnA, WnB, bn1, WeA, WeB, beA)


def _final_readout(G, Weo_p, Wr1, br1, Wr2_p, br2_p):
    def body(g_r, weo_r, wr1_r, br1_r, wr2_r, br2_r, o_r):
        seg = g_r[0] + g_r[1]
        pooled = _dot(seg, weo_r[...])
        r = jnp.maximum(_dot(pooled, wr1_r[...]) + br1_r[...], 0.0)
        o_r[...] = _dot(r, wr2_r[...]) + br2_r[...]

    return pl.pallas_call(
        body,
        out_shape=jax.ShapeDtypeStruct((NG, 8), _f32),
    )(G, Weo_p, Wr1, br1, Wr2_p, br2_p)


def _make_sc_round(last: bool):
    mesh = plsc.VectorSubcoreMesh(core_axis_name="c", subcore_axis_name="s")
    if last:
        out_type = jax.ShapeDtypeStruct((NC, NG, HP), _f32)
    else:
        out_type = (jax.ShapeDtypeStruct((E, HP), _f32),
                    jax.ShapeDtypeStruct((NC, N, HP), _f32))

    scratch = [
        pltpu.VMEM((K, HP), _f32),
        pltpu.VMEM((K, HP), _f32),
        pltpu.VMEM((K, HP), _f32),
        pltpu.VMEM((KQ, 80), jnp.int32),
        pltpu.VMEM((KQ, 80), jnp.int32),
        pltpu.SemaphoreType.DMA,
    ]
    if last:
        scratch += [
            pltpu.VMEM((KQ, 80), jnp.int32),
            pltpu.VMEM((N,), jnp.int32),
            pltpu.VMEM_SHARED((NG, HP), _f32),
        ]
    else:
        scratch += [
            pltpu.VMEM_SHARED((N, HP), _f32),
        ]

    def body(*refs):
        if last:
            (p_hbm, ha_hbm, hb_hbm, src2, dst2, batch_hbm, g_out,
             bufP, bufA, bufB, srcv, dstv, sem, gidv, batch_v, acc_sh) = refs
        else:
            (p_hbm, ha_hbm, hb_hbm, src2, dst2, e_out, m_out,
             bufP, bufA, bufB, srcv, dstv, sem, acc_sh) = refs

        c = lax.axis_index("c")
        s = lax.axis_index("s")
        wid = c * NS + s
        zero16 = jnp.zeros((16,), _f32)

        @plsc.parallel_loop(0, K, unroll=8)
        def _(r):
            for g in range(HP // 16):
                bufP[r, pl.ds(g * 16, 16)] = zero16

        if last:
            @pl.when(s == 0)
            def _():
                pltpu.sync_copy(bufP.at[pl.ds(0, NG)], acc_sh)
            pltpu.sync_copy(batch_hbm, batch_v)
        else:
            row0 = s * RPT
            pltpu.sync_copy(bufP, acc_sh.at[pl.ds(row0, K)])
            pltpu.sync_copy(bufP.at[pl.ds(0, RPT - K)],
                            acc_sh.at[pl.ds(row0 + K, RPT - K)])
        plsc.subcore_barrier()

        cvec = jnp.where(
            lax.broadcasted_iota(jnp.int32, (16,), 0) == (H - 64),
            1.0, 0.0).astype(_f32)

        def chunk(i, carry):
            base = wid * EPW + i * K
            qbase = (wid * EPW) // 80 + i * KQ
            pltpu.sync_copy(src2.at[pl.ds(qbase, KQ)], srcv)
            pltpu.sync_copy(dst2.at[pl.ds(qbase, KQ)], dstv)
            handles = [pltpu.async_copy(p_hbm.at[pl.ds(base, K)], bufP, sem)]
            for q in range(KQ):
                handles.append(pltpu.async_copy(
                    ha_hbm.at[srcv.at[q]], bufA.at[pl.ds(q * 80, 80)], sem))
                handles.append(pltpu.async_copy(
                    hb_hbm.at[dstv.at[q]], bufB.at[pl.ds(q * 80, 80)], sem))
            if last:
                for q in range(KQ):
                    for o in range(80 // 16):
                        sv = srcv[q, pl.ds(o * 16, 16)]
                        gidv[q, pl.ds(o * 16, 16)] = plsc.load_gather(
                            batch_v, [sv])
            for hnd in handles:
                hnd.wait()

            @plsc.parallel_loop(0, K, unroll=4)
            def _(r):
                for g in range(HP // 16):
                    sl = pl.ds(g * 16, 16)
                    t = jnp.maximum(bufA[r, sl] + bufB[r, sl] + bufP[r, sl],
                                    0.0)
                    if last and g == (HP // 16) - 1:
                        t = t + cvec
                    bufP[r, sl] = t

            if last:
                for q in range(KQ):
                    pltpu.sync_copy(bufP.at[pl.ds(q * 80, 80)],
                                    acc_sh.at[gidv.at[q]], add=True)
            else:
                pltpu.sync_copy(bufP, e_out.at[pl.ds(base, K)])
                for q in range(KQ):
                    pltpu.sync_copy(bufP.at[pl.ds(q * 80, 80)],
                                    acc_sh.at[dstv.at[q]], add=True)
            return carry

        lax.fori_loop(0, NCHUNK, chunk, 0)

        plsc.subcore_barrier()
        if last:
            @pl.when(s == 0)
            def _():
                pltpu.sync_copy(acc_sh, g_out.at[c])
        else:
            row0 = s * RPT
            pltpu.sync_copy(acc_sh.at[pl.ds(row0, RPT)],
                            m_out.at[c, pl.ds(row0, RPT)])

    return pl.kernel(body, out_type=out_type, mesh=mesh,
                     scratch_types=scratch)


_sc_round = _make_sc_round(last=False)
_sc_round_last = _make_sc_round(last=True)


def kernel(x, edge_attr, edge_index, batch, Wex, bex, Wee, bee, Wni, bni,
           Wei, bei, We, be, Wn, bn, Weo, beo, Wr1, br1, Wr2, br2):
    src2 = edge_index[0].reshape(E // 80, 80)
    dst2 = edge_index[1].reshape(E // 80, 80)

    def padw(w, r, c):
        return jnp.zeros((r, c), _f32).at[: w.shape[0], : w.shape[1]].set(w)

    def padb(b, c):
        return jnp.zeros((1, c), _f32).at[0, : b.shape[0]].set(b)

    Wni_p = padw(Wni, 128, HP)
    bni_p = padb(bni, HP)
    Wei_p = padw(Wei, 128, HP)
    bei_p = padb(bei, HP)
    WeA = [padw(We[l, :H, :], HP, HP) for l in range(3)]
    WeB = [padw(We[l, H:2 * H, :], HP, HP) for l in range(3)]
    WeC = [padw(We[l, 2 * H:, :], HP, HP) for l in range(3)]
    beA = [padb(be[l], HP) for l in range(3)]
    WnA = [padw(Wn[l, :H, :], HP, HP) for l in range(2)]
    WnB = [padw(Wn[l, H:, :], HP, HP) for l in range(2)]
    bnp = [padb(bn[l], HP) for l in range(2)]
    Weo_p = padw(Weo, HP, DOUT).at[H, :].set(beo)
    Wr2_p = padw(Wr2, DOUT, 8)
    br2_p = padb(br2, 8)

    h0, hA0, hB0 = _node_prologue(x, Wex, bex.reshape(1, 128), Wni_p, bni_p,
                                  WeA[0], WeB[0], beA[0])
    p0 = _edge_prologue(edge_attr, Wee, bee.reshape(1, 128), Wei_p, bei_p,
                        WeC[0])
    e1, m0 = _sc_round(p0, hA0, hB0, src2, dst2)
    h1, hA1, hB1 = _node_update(h0, m0, WnA[0], WnB[0], bnp[0],
                                WeA[1], WeB[1], beA[1])
    p1 = _p_matmul(e1, WeC[1])
    e2, m1 = _sc_round(p1, hA1, hB1, src2, dst2)
    _, hA2, hB2 = _node_update(h1, m1, WnA[1], WnB[1], bnp[1],
                               WeA[2], WeB[2], beA[2])
    p2 = _p_matmul(e2, WeC[2])
    G = _sc_round_last(p2, hA2, hB2, src2, dst2, batch)
    out8 = _final_readout(G, Weo_p, Wr1, br1.reshape(1, DOUT), Wr2_p, br2_p)
    return out8[:, :1]


# speedup vs baseline: 4.8327x; 4.8327x over previous
import functools

import jax
import jax.numpy as jnp
from jax import lax
from jax.experimental import pallas as pl
from jax.experimental.pallas import tpu as pltpu
from jax.experimental.pallas import tpu_sc as plsc

N, E, NG = 10000, 320000, 128
H, HP = 76, 128
DOUT = 64
NGRP = 5
NC, NS, NW = 2, 16, 32
K = 128
KG = K // 128
NJ = E // K
NT = -(-NJ // NW)
RPT = 624

_f32 = jnp.float32


def _dot(a, b):
    return jnp.dot(a, b, preferred_element_type=_f32)


def _node_prologue(x, Wex, bex, Wni_p, bni_p, WeA, WeB, beA):
    BN = 2000

    def body(x_r, wex_r, bex_r, wni_r, bni_r, wea_r, web_r, bea_r,
             h_o, ha_o, hb_o):
        hx = jnp.maximum(_dot(x_r[...], wex_r[...]) + bex_r[...], 0.0)
        h = jnp.maximum(_dot(hx, wni_r[...]) + bni_r[...], 0.0)
        h_o[...] = h
        ha_o[...] = _dot(h, wea_r[...]) + bea_r[...]
        hb_o[...] = _dot(h, web_r[...])

    full = lambda s: pl.BlockSpec(s, lambda i: (0,) * len(s))
    return pl.pallas_call(
        body,
        grid=(N // BN,),
        in_specs=[
            pl.BlockSpec((BN, 128), lambda i: (i, 0)),
            full((128, 128)), full((1, 128)),
            full((128, HP)), full((1, HP)),
            full((HP, HP)), full((HP, HP)), full((1, HP)),
        ],
        out_specs=[pl.BlockSpec((BN, HP), lambda i: (i, 0))] * 3,
        out_shape=[jax.ShapeDtypeStruct((N, HP), _f32)] * 3,
        compiler_params=pltpu.CompilerParams(dimension_semantics=("parallel",)),
    )(x, Wex, bex, Wni_p, bni_p, WeA, WeB, beA)


def _edge_prologue(edge_attr, Wee, bee, Wei_p, bei_p, WeC0):
    BE = 8000

    def body(ea_r, wee_r, bee_r, wei_r, bei_r, wc_r, p_o):
        he = jnp.maximum(_dot(ea_r[...], wee_r[...]) + bee_r[...], 0.0)
        e0 = jnp.maximum(_dot(he, wei_r[...]) + bei_r[...], 0.0)
        p_o[...] = _dot(e0, wc_r[...])

    full = lambda s: pl.BlockSpec(s, lambda i: (0,) * len(s))
    return pl.pallas_call(
        body,
        grid=(E // BE,),
        in_specs=[
            pl.BlockSpec((BE, 16), lambda i: (i, 0)),
            full((16, 128)), full((1, 128)),
            full((128, HP)), full((1, HP)), full((HP, HP)),
        ],
        out_specs=pl.BlockSpec((BE, HP), lambda i: (i, 0)),
        out_shape=jax.ShapeDtypeStruct((E, HP), _f32),
        compiler_params=pltpu.CompilerParams(dimension_semantics=("parallel",)),
    )(edge_attr, Wee, bee, Wei_p, bei_p, WeC0)


def _p_matmul(e, WeC):
    BE = 8000

    def body(e_r, w_r, p_o):
        p_o[...] = _dot(e_r[...], w_r[...])

    return pl.pallas_call(
        body,
        grid=(E // BE,),
        in_specs=[
            pl.BlockSpec((BE, HP), lambda i: (i, 0)),
            pl.BlockSpec((HP, HP), lambda i: (0, 0)),
        ],
        out_specs=pl.BlockSpec((BE, HP), lambda i: (i, 0)),
        out_shape=jax.ShapeDtypeStruct((E, HP), _f32),
        compiler_params=pltpu.CompilerParams(dimension_semantics=("parallel",)),
    )(e, WeC)


def _node_update(h, m, WnA, WnB, bn1, WeA, WeB, beA):
    BN = 2000

    def body(h_r, m_r, wna_r, wnb_r, bn_r, wea_r, web_r, bea_r,
             h_o, ha_o, hb_o):
        msum = m_r[0] + m_r[1]
        hn = jnp.maximum(
            _dot(h_r[...], wna_r[...]) + _dot(msum, wnb_r[...]) + bn_r[...],
            0.0)
        h_o[...] = hn
        ha_o[...] = _dot(hn, wea_r[...]) + bea_r[...]
        hb_o[...] = _dot(hn, web_r[...])

    full = lambda s: pl.BlockSpec(s, lambda i: (0,) * len(s))
    return pl.pallas_call(
        body,
        grid=(N // BN,),
        in_specs=[
            pl.BlockSpec((BN, HP), lambda i: (i, 0)),
            pl.BlockSpec((NC, BN, HP), lambda i: (0, i, 0)),
            full((HP, HP)), full((HP, HP)), full((1, HP)),
            full((HP, HP)), full((HP, HP)), full((1, HP)),
        ],
        out_specs=[pl.BlockSpec((BN, HP), lambda i: (i, 0))] * 3,
        out_shape=[jax.ShapeDtypeStruct((N, HP), _f32)] * 3,
        compiler_params=pltpu.CompilerParams(dimension_semantics=("parallel",)),
    )(h, m, WnA, WnB, bn1, WeA, WeB, beA)


def _final_readout(G, Weo_p, Wr1, br1, Wr2_p, br2_p):
    def body(g_r, weo_r, wr1_r, br1_r, wr2_r, br2_r, o_r):
        seg = g_r[0] + g_r[1]
        pooled = _dot(seg, weo_r[...])
        r = jnp.maximum(_dot(pooled, wr1_r[...]) + br1_r[...], 0.0)
        o_r[...] = _dot(r, wr2_r[...]) + br2_r[...]

    return pl.pallas_call(
        body,
        out_shape=jax.ShapeDtypeStruct((NG, 8), _f32),
    )(G, Weo_p, Wr1, br1, Wr2_p, br2_p)


_MESH = plsc.VectorSubcoreMesh(core_axis_name="c", subcore_axis_name="s")
_SC_PARAMS = pltpu.CompilerParams(needs_layout_passes=False)
NT2 = (NT + 1) // 2


def _zero_fill(buf):
    zero16 = jnp.zeros((16,), _f32)

    @plsc.parallel_loop(0, K, unroll=8)
    def _(r):
        for g in range(HP // 16):
            buf[r, pl.ds(g * 16, 16)] = zero16


def _seed_node_acc(s, buf, acc_sh):
    row0 = s * RPT
    for t0 in range(RPT // K):
        pltpu.sync_copy(buf, acc_sh.at[pl.ds(row0 + t0 * K, K)])
    if RPT % K:
        pltpu.sync_copy(buf.at[pl.ds(0, RPT % K)],
                        acc_sh.at[pl.ds(row0 + (RPT // K) * K, RPT % K)])

    @pl.when(s == NS - 1)
    def _():
        pltpu.sync_copy(buf.at[pl.ds(0, N - NS * RPT)],
                        acc_sh.at[pl.ds(NS * RPT, N - NS * RPT)])


def _flush_node_acc(c, s, acc_sh, m_out):
    row0 = s * RPT
    pltpu.sync_copy(acc_sh.at[pl.ds(row0, RPT)],
                    m_out.at[c, pl.ds(row0, RPT)])

    @pl.when(s == NS - 1)
    def _():
        pltpu.sync_copy(acc_sh.at[pl.ds(NS * RPT, N - NS * RPT)],
                        m_out.at[c, pl.ds(NS * RPT, N - NS * RPT)])


def _make_sc_gather(last: bool):
    if last:
        out_type = jax.ShapeDtypeStruct((NC, NG, HP), _f32)
    else:
        out_type = jax.ShapeDtypeStruct((E, HP), _f32)

    scratch = [
        pltpu.VMEM((K, HP), _f32), pltpu.VMEM((K, HP), _f32),
        pltpu.VMEM((K, HP), _f32), pltpu.VMEM((K, HP), _f32),
        pltpu.VMEM((K, HP), _f32), pltpu.VMEM((K, HP), _f32),
        pltpu.VMEM((1, 128), jnp.int32), pltpu.VMEM((1, 128), jnp.int32),
        pltpu.VMEM((1, 128), jnp.int32), pltpu.VMEM((1, 128), jnp.int32),
        pltpu.SemaphoreType.DMA, pltpu.SemaphoreType.DMA,
    ]
    if last:
        scratch += [
            pltpu.VMEM((1, 128), jnp.int32), pltpu.VMEM((1, 128), jnp.int32),
            pltpu.VMEM((N,), jnp.int32),
            pltpu.VMEM_SHARED((NG, HP), _f32),
        ]

    def body(*refs):
        if last:
            (p_hbm, ha_hbm, hb_hbm, src1, dst1, batch_hbm, g_out,
             bP0, bP1, bA0, bA1, bB0, bB1, sv0, sv1, dv0, dv1,
             sem0, sem1, gv0, gv1, batch_v, acc_sh) = refs
        else:
            (p_hbm, ha_hbm, hb_hbm, src1, dst1, e_out,
             bP0, bP1, bA0, bA1, bB0, bB1, sv0, sv1, dv0, dv1,
             sem0, sem1) = refs
        bufP, bufA, bufB = (bP0, bP1), (bA0, bA1), (bB0, bB1)
        srcv, dstv, sem = (sv0, sv1), (dv0, dv1), (sem0, sem1)
        if last:
            gidv = (gv0, gv1)

        c = lax.axis_index("c")
        s = lax.axis_index("s")
        wid = c * NS + s

        if last:
            _zero_fill(bP0)

            @pl.when(s == 0)
            def _():
                pltpu.sync_copy(bP0.at[pl.ds(0, NG)], acc_sh)
            pltpu.sync_copy(batch_hbm, batch_v)
            plsc.subcore_barrier()

        def jn(t):
            return wid + NW * t

        def descs(q, j):
            base = j * K
            return (
                pltpu.make_async_copy(p_hbm.at[pl.ds(base, K)], bufP[q],
                                      sem[q]),
                pltpu.make_async_copy(ha_hbm.at[srcv[q].at[0]], bufA[q],
                                      sem[q]),
                pltpu.make_async_copy(hb_hbm.at[dstv[q].at[0]], bufB[q],
                                      sem[q]),
            )

        def start(q, j, ds_):
            pltpu.sync_copy(src1.at[pl.ds(j * K, K)], srcv[q].at[0])
            pltpu.sync_copy(dst1.at[pl.ds(j * K, K)], dstv[q].at[0])
            for d in ds_:
                d.start()

        cvec = jnp.where(
            lax.broadcasted_iota(jnp.int32, (16,), 0) == (H - 64),
            1.0, 0.0).astype(_f32)

        def compute(par, t):
            if last:
                for o in range(128 // 16):
                    sv = srcv[par][0, pl.ds(o * 16, 16)]
                    gidv[par][0, pl.ds(o * 16, 16)] = \
                        plsc.load_gather(batch_v, [sv])

            @plsc.parallel_loop(0, K, unroll=4)
            def _(r):
                for g in range(NGRP):
                    sl = pl.ds(g * 16, 16)
                    t2 = jnp.maximum(
                        bufA[par][r, sl] + bufB[par][r, sl]
                        + bufP[par][r, sl], 0.0)
                    if last and g == NGRP - 1:
                        t2 = t2 + cvec
                    bufP[par][r, sl] = t2

            if last:
                pltpu.sync_copy(bufP[par], acc_sh.at[gidv[par].at[0]],
                                add=True)
            else:
                pltpu.sync_copy(bufP[par], e_out.at[pl.ds(jn(t) * K, K)])

        d0 = descs(0, jn(0))
        start(0, jn(0), d0)
        for d in d0:
            d.wait()

        def step(tt, carry):
            for par in (0, 1):
                t = 2 * tt + par
                jnext = jn(t + 1)
                dn = descs(1 - par, jnext)

                @pl.when(jnext < NJ)
                def _():
                    start(1 - par, jnext, dn)

                @pl.when((t < NT) & (jn(t) < NJ))
                def _():
                    compute(par, t)

                @pl.when(jnext < NJ)
                def _():
                    for d in dn:
                        d.wait()
            return carry

        lax.fori_loop(0, NT2, step, 0)

        if last:
            plsc.subcore_barrier()

            @pl.when(s == 0)
            def _():
                pltpu.sync_copy(acc_sh, g_out.at[c])

    return pl.kernel(body, out_type=out_type, mesh=_MESH,
                     scratch_types=scratch, compiler_params=_SC_PARAMS)


def _make_sc_scatter():
    out_type = jax.ShapeDtypeStruct((NC, N, HP), _f32)
    scratch = [
        pltpu.VMEM((K, HP), _f32), pltpu.VMEM((K, HP), _f32),
        pltpu.VMEM((1, 128), jnp.int32), pltpu.VMEM((1, 128), jnp.int32),
        pltpu.SemaphoreType.DMA, pltpu.SemaphoreType.DMA,
        pltpu.VMEM_SHARED((N, HP), _f32),
    ]

    def body(e_hbm, dst1, m_out, bE0, bE1, dv0, dv1, sem0, sem1, acc_sh):
        bufE, dstv, sem = (bE0, bE1), (dv0, dv1), (sem0, sem1)
        c = lax.axis_index("c")
        s = lax.axis_index("s")
        wid = c * NS + s

        _zero_fill(bE0)
        _seed_node_acc(s, bE0, acc_sh)
        plsc.subcore_barrier()

        def jn(t):
            return wid + NW * t

        def desc(q, j):
            return pltpu.make_async_copy(e_hbm.at[pl.ds(j * K, K)], bufE[q],
                                         sem[q])

        def start(q, j, d):
            pltpu.sync_copy(dst1.at[pl.ds(j * K, K)], dstv[q].at[0])
            d.start()

        d0 = desc(0, jn(0))
        start(0, jn(0), d0)
        d0.wait()

        def step(tt, carry):
            for par in (0, 1):
                t = 2 * tt + par
                jnext = jn(t + 1)
                dn = desc(1 - par, jnext)

                @pl.when(jnext < NJ)
                def _():
                    start(1 - par, jnext, dn)

                @pl.when((t < NT) & (jn(t) < NJ))
                def _():
                    pltpu.sync_copy(bufE[par], acc_sh.at[dstv[par].at[0]],
                                    add=True)

                @pl.when(jnext < NJ)
                def _():
                    dn.wait()
            return carry

        lax.fori_loop(0, NT2, step, 0)
        plsc.subcore_barrier()
        _flush_node_acc(c, s, acc_sh, m_out)

    return pl.kernel(body, out_type=out_type, mesh=_MESH,
                     scratch_types=scratch, compiler_params=_SC_PARAMS)


_sc_gather = _make_sc_gather(last=False)
_sc_scatter = _make_sc_scatter()
_sc_round_last = _make_sc_gather(last=True)


def kernel(x, edge_attr, edge_index, batch, Wex, bex, Wee, bee, Wni, bni,
           Wei, bei, We, be, Wn, bn, Weo, beo, Wr1, br1, Wr2, br2):
    src1 = edge_index[0]
    dst1 = edge_index[1]

    def padw(w, r, c):
        return jnp.zeros((r, c), _f32).at[: w.shape[0], : w.shape[1]].set(w)

    def padb(b, c):
        return jnp.zeros((1, c), _f32).at[0, : b.shape[0]].set(b)

    Wni_p = padw(Wni, 128, HP)
    bni_p = padb(bni, HP)
    Wei_p = padw(Wei, 128, HP)
    bei_p = padb(bei, HP)
    WeA = [padw(We[l, :H, :], HP, HP) for l in range(3)]
    WeB = [padw(We[l, H:2 * H, :], HP, HP) for l in range(3)]
    WeC = [padw(We[l, 2 * H:, :], HP, HP) for l in range(3)]
    beA = [padb(be[l], HP) for l in range(3)]
    WnA = [padw(Wn[l, :H, :], HP, HP) for l in range(2)]
    WnB = [padw(Wn[l, H:, :], HP, HP) for l in range(2)]
    bnp = [padb(bn[l], HP) for l in range(2)]
    Weo_p = padw(Weo, HP, DOUT).at[H, :].set(beo)
    Wr2_p = padw(Wr2, DOUT, 8)
    br2_p = padb(br2, 8)

    h0, hA0, hB0 = _node_prologue(x, Wex, bex.reshape(1, 128), Wni_p, bni_p,
                                  WeA[0], WeB[0], beA[0])
    p0 = _edge_prologue(edge_attr, Wee, bee.reshape(1, 128), Wei_p, bei_p,
                        WeC[0])

    stacked = (
        jnp.stack(WnA), jnp.stack(WnB), jnp.stack(bnp),
        jnp.stack(WeA[1:]), jnp.stack(WeB[1:]), jnp.stack(beA[1:]),
        jnp.stack(WeC[1:]),
    )

    def round_body(carry, ws):
        p, h, hA, hB = carry
        wna, wnb, bn_, wea, web, bea, wec = ws
        e_n = _sc_gather(p, hA, hB, src1, dst1)
        m = _sc_scatter(e_n, dst1)
        h_n, hA_n, hB_n = _node_update(h, m, wna, wnb, bn_, wea, web, bea)
        p_n = _p_matmul(e_n, wec)
        return (p_n, h_n, hA_n, hB_n), None

    (p2, _, hA2, hB2), _ = lax.scan(
        round_body, (p0, h0, hA0, hB0), stacked)
    G = _sc_round_last(p2, hA2, hB2, src1, dst1, batch)
    out8 = _final_readout(G, Weo_p, Wr1, br1.reshape(1, DOUT), Wr2_p, br2_p)
    return out8[:, :1]
